# baseline (speedup 1.0000x reference)
import jax
import jax.numpy as jnp
from jax.experimental import pallas as pl
from jax.experimental.pallas import tpu as pltpu

_BATCH = 16384
_F = 100
_NFIELD = 26
_K = 128
_BK = 16384


def _fm_kernel(x_ref, fi_ref, w_ref, v_ref, o_ref):
    fi = fi_ref[...]
    onehot = (fi[:, None] ==
              jax.lax.broadcasted_iota(jnp.int32, (_F, _NFIELD), 1)
              ).astype(jnp.float32)
    e = jnp.dot(onehot, v_ref[...],
                preferred_element_type=jnp.float32)
    g = jnp.dot(e, e.T, preferred_element_type=jnp.float32)
    eye = (jax.lax.broadcasted_iota(jnp.int32, (_F, _F), 0) ==
           jax.lax.broadcasted_iota(jnp.int32, (_F, _F), 1))
    g0 = jnp.where(eye, jnp.float32(0.0), g)
    wv = w_ref[...][:, None]

    gp = jnp.concatenate(
        [g0, jnp.zeros((_F, _K - _F), jnp.float32)], axis=1)
    x = x_ref[...]
    h = jnp.dot(x, gp, preferred_element_type=jnp.float32)
    quad = jnp.sum(x * h[:, :_F], axis=1, keepdims=True)
    lin = jnp.dot(x, wv, preferred_element_type=jnp.float32)
    o_ref[...] = lin + 0.5 * quad


def kernel(inputs, field_index, w, V):
    grid = (_BATCH // _BK,)
    out = pl.pallas_call(
        _fm_kernel,
        grid=grid,
        in_specs=[
            pl.BlockSpec((_BK, _F), lambda i: (i, 0)),
            pl.BlockSpec((_F,), lambda i: (0,)),
            pl.BlockSpec((_F,), lambda i: (0,)),
            pl.BlockSpec((_NFIELD, _K), lambda i: (0, 0)),
        ],
        out_specs=pl.BlockSpec((_BK, 1), lambda i: (i, 0)),
        out_shape=jax.ShapeDtypeStruct((_BATCH, 1), jnp.float32),
        compiler_params=pltpu.CompilerParams(
            dimension_semantics=("parallel",)),
    )(inputs, field_index, w, V)
    return out


# speedup vs baseline: 1.0807x; 1.0807x over previous
import jax
import jax.numpy as jnp
from jax.experimental import pallas as pl
from jax.experimental.pallas import tpu as pltpu

_BATCH = 16384
_F = 100
_NFIELD = 26
_K = 128
_BK = 8192


def _fm_kernel(x_ref, fi_ref, w_ref, v_ref, o_ref):
    fi = fi_ref[...]
    onehot = (fi[:, None] ==
              jax.lax.broadcasted_iota(jnp.int32, (_F, _NFIELD), 1)
              ).astype(jnp.float32)
    e = jnp.dot(onehot, v_ref[...],
                preferred_element_type=jnp.float32)
    g = jnp.dot(e, e.T, preferred_element_type=jnp.float32)
    eye = (jax.lax.broadcasted_iota(jnp.int32, (_F, _F), 0) ==
           jax.lax.broadcasted_iota(jnp.int32, (_F, _F), 1))
    g0 = jnp.where(eye, jnp.float32(0.0), g)
    wv = w_ref[...][:, None]

    gp = jnp.concatenate(
        [g0, jnp.zeros((_F, _K - _F), jnp.float32)], axis=1)
    x = x_ref[...]
    h = jnp.dot(x, gp, preferred_element_type=jnp.float32)
    quad = jnp.sum(x * h[:, :_F], axis=1, keepdims=True)
    lin = jnp.dot(x, wv, preferred_element_type=jnp.float32)
    o_ref[...] = lin + 0.5 * quad


def kernel(inputs, field_index, w, V):
    grid = (_BATCH // _BK,)
    out = pl.pallas_call(
        _fm_kernel,
        grid=grid,
        in_specs=[
            pl.BlockSpec((_BK, _F), lambda i: (i, 0)),
            pl.BlockSpec((_F,), lambda i: (0,)),
            pl.BlockSpec((_F,), lambda i: (0,)),
            pl.BlockSpec((_NFIELD, _K), lambda i: (0, 0)),
        ],
        out_specs=pl.BlockSpec((_BK, 1), lambda i: (i, 0)),
        out_shape=jax.ShapeDtypeStruct((_BATCH, 1), jnp.float32),
        compiler_params=pltpu.CompilerParams(
            dimension_semantics=("parallel",)),
    )(inputs, field_index, w, V)
    return out
